# baseline (speedup 1.0000x reference)
import jax
import jax.numpy as jnp
from jax.experimental import pallas as pl
from jax.experimental.pallas import tpu as pltpu

_E = 8
_TOPK = 2


def _pool_kernel(x_ref, out_ref):
    ci = pl.program_id(1)
    h = x_ref.shape[2]
    w = x_ref.shape[3]
    nchunks = pl.num_programs(1)
    scale = 1.0 / (h * nchunks * w)
    s = (jnp.sum(x_ref[0], axis=(1, 2)) * scale).reshape(1, -1)

    @pl.when(ci == 0)
    def _():
        out_ref[0] = s

    @pl.when(ci > 0)
    def _():
        out_ref[0] = out_ref[0] + s


def _gating_kernel(gx_ref, wg_ref, wf_ref, bias_ref,
                   wcomb_ref, bcomb_ref, loss_ref):
    gx = gx_ref[...]
    wg = wg_ref[...]
    logits = jnp.dot(gx, wg, preferred_element_type=jnp.float32)
    z = logits - jnp.max(logits, axis=1, keepdims=True)
    ez = jnp.exp(z)
    p = ez / jnp.sum(ez, axis=1, keepdims=True)

    iota = jax.lax.broadcasted_iota(jnp.int32, p.shape, 1)
    m1 = jnp.max(p, axis=1, keepdims=True)
    e1 = jnp.min(jnp.where(p == m1, iota, _E), axis=1, keepdims=True)
    p2 = jnp.where(iota == e1, -jnp.inf, p)
    m2 = jnp.max(p2, axis=1, keepdims=True)
    e2 = jnp.min(jnp.where(p2 == m2, iota, _E), axis=1, keepdims=True)
    denom = m1 + m2 + 1e-6
    gates = (jnp.where(iota == e1, m1, 0.0)
             + jnp.where(iota == e2, m2, 0.0)) / denom

    def _cv_sq(v):
        mean = jnp.sum(v, keepdims=True) / _E
        var = jnp.sum((v - mean) ** 2, keepdims=True) / (_E - 1)
        return var / (mean * mean + 1e-10)

    importance = jnp.sum(gates, axis=0, keepdims=True)
    load = jnp.sum((gates > 0).astype(jnp.float32), axis=0, keepdims=True)
    loss_ref[...] = (_cv_sq(importance) + _cv_sq(load)) * 0.01

    wcomb_ref[...] = jnp.dot(gates, wf_ref[...],
                             preferred_element_type=jnp.float32)
    bcomb_ref[...] = jnp.dot(gates, bias_ref[...],
                             preferred_element_type=jnp.float32)


def _conv_kernel(w_ref, x_ref, b_ref, out_ref):
    out_ref[0] = (jnp.dot(w_ref[0], x_ref[0],
                          preferred_element_type=jnp.float32)
                  + b_ref[0])


def kernel(x, train, w_gate, w_noise, W, b):
    del train, w_noise
    B, IC, H, Wd = x.shape
    E, OC = W.shape[0], W.shape[1]
    OH, OW = H // 2, Wd // 2
    S = OH * OW
    K9 = 9 * IC

    hchunks = 4
    gate_x = pl.pallas_call(
        _pool_kernel,
        grid=(B, hchunks),
        in_specs=[pl.BlockSpec((1, IC, H // hchunks, Wd),
                               lambda bi, ci: (bi, 0, ci, 0))],
        out_specs=pl.BlockSpec((1, 1, IC), lambda bi, ci: (bi, 0, 0)),
        out_shape=jax.ShapeDtypeStruct((B, 1, IC), jnp.float32),
        compiler_params=pltpu.CompilerParams(
            dimension_semantics=("arbitrary", "arbitrary")),
    )(x)
    gate_x = gate_x.reshape(B, IC)

    W_flat = jnp.transpose(W, (0, 1, 3, 4, 2)).reshape(E, OC * K9)
    w_comb, b_comb, loss = pl.pallas_call(
        _gating_kernel,
        out_shape=(
            jax.ShapeDtypeStruct((B, OC * K9), jnp.float32),
            jax.ShapeDtypeStruct((B, OC), jnp.float32),
            jax.ShapeDtypeStruct((1, 1), jnp.float32),
        ),
    )(gate_x, w_gate, W_flat, b)

    x_pad = jnp.pad(x, ((0, 0), (0, 0), (1, 1), (1, 1)))
    taps = [x_pad[:, :, ky::2, kx::2][:, :, :OH, :OW]
            for ky in range(3) for kx in range(3)]
    x_taps = jnp.stack(taps, axis=1).reshape(B, K9, S)

    w_comb = w_comb.reshape(B, OC, K9)
    b_comb = b_comb.reshape(B, OC, 1)

    schunks = 7
    SB = S // schunks
    y = pl.pallas_call(
        _conv_kernel,
        grid=(B, schunks),
        in_specs=[
            pl.BlockSpec((1, OC, K9), lambda bi, ci: (bi, 0, 0)),
            pl.BlockSpec((1, K9, SB), lambda bi, ci: (bi, 0, ci)),
            pl.BlockSpec((1, OC, 1), lambda bi, ci: (bi, 0, 0)),
        ],
        out_specs=pl.BlockSpec((1, OC, SB), lambda bi, ci: (bi, 0, ci)),
        out_shape=jax.ShapeDtypeStruct((B, OC, S), jnp.float32),
        compiler_params=pltpu.CompilerParams(
            dimension_semantics=("parallel", "parallel")),
    )(w_comb, x_taps, b_comb)

    y = y.reshape(B, OC, OH, OW)
    return y, loss.reshape(())


# speedup vs baseline: 9.8907x; 9.8907x over previous
import jax
import jax.numpy as jnp
from jax.experimental import pallas as pl
from jax.experimental.pallas import tpu as pltpu

_E = 8
_TOPK = 2


def _pool_kernel(p00_ref, p01_ref, p10_ref, p11_ref, out_ref):
    ci = pl.program_id(1)
    npx = 2 * p00_ref.shape[2]
    scale = 1.0 / (npx * npx)
    s = (jnp.sum(p00_ref[0], axis=(0, 1)) + jnp.sum(p01_ref[0], axis=(0, 1))
         + jnp.sum(p10_ref[0], axis=(0, 1))
         + jnp.sum(p11_ref[0], axis=(0, 1))).reshape(1, -1) * scale

    @pl.when(ci == 0)
    def _():
        out_ref[0] = s

    @pl.when(ci > 0)
    def _():
        out_ref[0] = out_ref[0] + s


def _gating_kernel(gx_ref, wg_ref, wf_ref, bias_ref,
                   wcomb_ref, bcomb_ref, loss_ref):
    gx = gx_ref[...]
    wg = wg_ref[...]
    logits = jnp.dot(gx, wg, preferred_element_type=jnp.float32)
    z = logits - jnp.max(logits, axis=1, keepdims=True)
    ez = jnp.exp(z)
    p = ez / jnp.sum(ez, axis=1, keepdims=True)

    iota = jax.lax.broadcasted_iota(jnp.int32, p.shape, 1)
    m1 = jnp.max(p, axis=1, keepdims=True)
    e1 = jnp.min(jnp.where(p == m1, iota, _E), axis=1, keepdims=True)
    p2 = jnp.where(iota == e1, -jnp.inf, p)
    m2 = jnp.max(p2, axis=1, keepdims=True)
    e2 = jnp.min(jnp.where(p2 == m2, iota, _E), axis=1, keepdims=True)
    denom = m1 + m2 + 1e-6
    gates = (jnp.where(iota == e1, m1, 0.0)
             + jnp.where(iota == e2, m2, 0.0)) / denom

    def _cv_sq(v):
        mean = jnp.sum(v, keepdims=True) / _E
        var = jnp.sum((v - mean) ** 2, keepdims=True) / (_E - 1)
        return var / (mean * mean + 1e-10)

    importance = jnp.sum(gates, axis=0, keepdims=True)
    load = jnp.sum((gates > 0).astype(jnp.float32), axis=0, keepdims=True)
    loss_ref[...] = (_cv_sq(importance) + _cv_sq(load)) * 0.01

    wcomb_ref[...] = jnp.dot(gates, wf_ref[...],
                             preferred_element_type=jnp.float32)
    bcomb_ref[...] = jnp.dot(gates, bias_ref[...],
                             preferred_element_type=jnp.float32)


def _shift_col(p):
    return jnp.concatenate(
        [jnp.zeros((p.shape[0], 1, p.shape[2]), p.dtype), p[:, :-1, :]],
        axis=1)


def _conv_kernel(w_ref, p00_ref, p01_ref, p10_ref, p11_ref,
                 p10m_ref, p11m_ref, b_ref, out_ref):
    R, OW, OC = out_ref.shape[1], out_ref.shape[2], out_ref.shape[3]
    p00 = p00_ref[0]
    p01 = p01_ref[0]
    p10 = p10_ref[0]
    p11 = p11_ref[0]
    p10m = p10m_ref[0]
    p11m = p11m_ref[0]
    taps = (
        (_shift_col(p11m), 0), (p10m, 1), (p11m, 2),
        (_shift_col(p01), 3), (p00, 4), (p01, 5),
        (_shift_col(p11), 6), (p10, 7), (p11, 8),
    )
    acc = None
    for patch, t in taps:
        patch = patch.reshape(R * OW, patch.shape[2])
        d = jnp.dot(patch, w_ref[0, t], preferred_element_type=jnp.float32)
        acc = d if acc is None else acc + d
    out_ref[0] = (acc + b_ref[0]).reshape(R, OW, OC)


def kernel(x, train, w_gate, w_noise, W, b):
    del train, w_noise
    B, IC, H, Wd = x.shape
    E, OC = W.shape[0], W.shape[1]
    OH, OW = H // 2, Wd // 2

    xr = x.reshape(B, IC, OH, 2, OW, 2).transpose(0, 3, 5, 2, 4, 1)
    p00 = xr[:, 0, 0]
    p01 = xr[:, 0, 1]
    p10 = xr[:, 1, 0]
    p11 = xr[:, 1, 1]
    p10m = jnp.pad(p10, ((0, 0), (1, 0), (0, 0), (0, 0)))[:, :OH]
    p11m = jnp.pad(p11, ((0, 0), (1, 0), (0, 0), (0, 0)))[:, :OH]

    rchunks = 7
    R = OH // rchunks
    phase_spec = pl.BlockSpec((1, R, OW, IC), lambda bi, ci: (bi, ci, 0, 0))

    gate_x = pl.pallas_call(
        _pool_kernel,
        grid=(B, rchunks),
        in_specs=[phase_spec] * 4,
        out_specs=pl.BlockSpec((1, 1, IC), lambda bi, ci: (bi, 0, 0)),
        out_shape=jax.ShapeDtypeStruct((B, 1, IC), jnp.float32),
        compiler_params=pltpu.CompilerParams(
            dimension_semantics=("arbitrary", "arbitrary")),
    )(p00, p01, p10, p11)
    gate_x = gate_x.reshape(B, IC)

    W_flat = jnp.transpose(W, (0, 3, 4, 2, 1)).reshape(E, 9 * IC * OC)
    w_comb, b_comb, loss = pl.pallas_call(
        _gating_kernel,
        out_shape=(
            jax.ShapeDtypeStruct((B, 9 * IC * OC), jnp.float32),
            jax.ShapeDtypeStruct((B, OC), jnp.float32),
            jax.ShapeDtypeStruct((1, 1), jnp.float32),
        ),
    )(gate_x, w_gate, W_flat, b)

    w_comb = w_comb.reshape(B, 9, IC, OC)
    b_comb = b_comb.reshape(B, 1, OC)

    y = pl.pallas_call(
        _conv_kernel,
        grid=(B, rchunks),
        in_specs=[
            pl.BlockSpec((1, 9, IC, OC), lambda bi, ci: (bi, 0, 0, 0)),
            phase_spec, phase_spec, phase_spec, phase_spec,
            phase_spec, phase_spec,
            pl.BlockSpec((1, 1, OC), lambda bi, ci: (bi, 0, 0)),
        ],
        out_specs=pl.BlockSpec((1, R, OW, OC), lambda bi, ci: (bi, ci, 0, 0)),
        out_shape=jax.ShapeDtypeStruct((B, OH, OW, OC), jnp.float32),
        compiler_params=pltpu.CompilerParams(
            dimension_semantics=("parallel", "parallel")),
    )(w_comb, p00, p01, p10, p11, p10m, p11m, b_comb)

    y = jnp.transpose(y, (0, 3, 1, 2))
    return y, loss.reshape(())


# speedup vs baseline: 21.2788x; 2.1514x over previous
import jax
import jax.numpy as jnp
from jax.experimental import pallas as pl
from jax.experimental.pallas import tpu as pltpu

_E = 8
_TOPK = 2


def _pool_kernel(x_ref, out_ref):
    ci = pl.program_id(1)
    w = x_ref.shape[3]
    scale = 1.0 / (w * w)
    s = (jnp.sum(x_ref[0], axis=(1, 2)) * scale).reshape(-1, 1)

    @pl.when(ci == 0)
    def _():
        out_ref[0] = s

    @pl.when(ci > 0)
    def _():
        out_ref[0] = out_ref[0] + s


def _gating_kernel(gx_ref, wg_ref, wf_ref, bias_ref,
                   wcomb_ref, bcomb_ref, loss_ref):
    gx = gx_ref[...]
    wg = wg_ref[...]
    logits = jnp.dot(gx, wg, preferred_element_type=jnp.float32)
    z = logits - jnp.max(logits, axis=1, keepdims=True)
    ez = jnp.exp(z)
    p = ez / jnp.sum(ez, axis=1, keepdims=True)

    iota = jax.lax.broadcasted_iota(jnp.int32, p.shape, 1)
    m1 = jnp.max(p, axis=1, keepdims=True)
    e1 = jnp.min(jnp.where(p == m1, iota, _E), axis=1, keepdims=True)
    p2 = jnp.where(iota == e1, -jnp.inf, p)
    m2 = jnp.max(p2, axis=1, keepdims=True)
    e2 = jnp.min(jnp.where(p2 == m2, iota, _E), axis=1, keepdims=True)
    denom = m1 + m2 + 1e-6
    gates = (jnp.where(iota == e1, m1, 0.0)
             + jnp.where(iota == e2, m2, 0.0)) / denom

    def _cv_sq(v):
        mean = jnp.sum(v, keepdims=True) / _E
        var = jnp.sum((v - mean) ** 2, keepdims=True) / (_E - 1)
        return var / (mean * mean + 1e-10)

    importance = jnp.sum(gates, axis=0, keepdims=True)
    load = jnp.sum((gates > 0).astype(jnp.float32), axis=0, keepdims=True)
    loss_ref[...] = (_cv_sq(importance) + _cv_sq(load)) * 0.01

    wcomb_ref[...] = jnp.dot(gates, wf_ref[...],
                             preferred_element_type=jnp.float32)
    bcomb_ref[...] = jnp.dot(gates, bias_ref[...],
                             preferred_element_type=jnp.float32)


def _shift_col(p):
    return jnp.concatenate(
        [jnp.zeros((p.shape[0], 1, p.shape[2]), p.dtype), p[:, :-1, :]],
        axis=1)


def _shift_row(p, prev_block):
    ci = pl.program_id(1)
    prev_row = prev_block[-1:, :, :]
    prev_row = jnp.where(ci == 0, jnp.zeros_like(prev_row), prev_row)
    return jnp.concatenate([prev_row, p[:-1, :, :]], axis=0)


def _conv_kernel(w_ref, p00_ref, p01_ref, p10_ref, p11_ref,
                 p10h_ref, p11h_ref, b_ref, out_ref):
    R, OW, OC = out_ref.shape[1], out_ref.shape[2], out_ref.shape[3]
    p00 = p00_ref[0]
    p01 = p01_ref[0]
    p10 = p10_ref[0]
    p11 = p11_ref[0]
    p10m = _shift_row(p10, p10h_ref[0])
    p11m = _shift_row(p11, p11h_ref[0])
    taps = (
        (_shift_col(p11m), 0), (p10m, 1), (p11m, 2),
        (_shift_col(p01), 3), (p00, 4), (p01, 5),
        (_shift_col(p11), 6), (p10, 7), (p11, 8),
    )
    acc = None
    for patch, t in taps:
        patch = patch.reshape(R * OW, patch.shape[2])
        d = jnp.dot(patch, w_ref[0, t], preferred_element_type=jnp.float32)
        acc = d if acc is None else acc + d
    out_ref[0] = (acc + b_ref[0]).reshape(R, OW, OC)


def kernel(x, train, w_gate, w_noise, W, b):
    del train, w_noise
    B, IC, H, Wd = x.shape
    E, OC = W.shape[0], W.shape[1]
    OH, OW = H // 2, Wd // 2

    hchunks = 4
    gate_x = pl.pallas_call(
        _pool_kernel,
        grid=(B, hchunks),
        in_specs=[pl.BlockSpec((1, IC, H // hchunks, Wd),
                               lambda bi, ci: (bi, 0, ci, 0))],
        out_specs=pl.BlockSpec((1, IC, 1), lambda bi, ci: (bi, 0, 0)),
        out_shape=jax.ShapeDtypeStruct((B, IC, 1), jnp.float32),
        compiler_params=pltpu.CompilerParams(
            dimension_semantics=("arbitrary", "arbitrary")),
    )(x)
    gate_x = gate_x.reshape(B, IC)

    W_flat = jnp.transpose(W, (0, 3, 4, 2, 1)).reshape(E, 9 * IC * OC)
    w_comb, b_comb, loss = pl.pallas_call(
        _gating_kernel,
        out_shape=(
            jax.ShapeDtypeStruct((B, 9 * IC * OC), jnp.float32),
            jax.ShapeDtypeStruct((B, OC), jnp.float32),
            jax.ShapeDtypeStruct((1, 1), jnp.float32),
        ),
    )(gate_x, w_gate, W_flat, b)

    w_comb = w_comb.astype(jnp.bfloat16).reshape(B, 9, IC, OC)
    b_comb = b_comb.reshape(B, 1, OC)

    xr = (x.astype(jnp.bfloat16)
          .reshape(B, IC, OH, 2, OW, 2).transpose(0, 3, 5, 2, 4, 1))
    p00 = xr[:, 0, 0]
    p01 = xr[:, 0, 1]
    p10 = xr[:, 1, 0]
    p11 = xr[:, 1, 1]

    rchunks = 7
    R = OH // rchunks
    phase_spec = pl.BlockSpec((1, R, OW, IC), lambda bi, ci: (bi, ci, 0, 0))
    halo_spec = pl.BlockSpec(
        (1, R, OW, IC),
        lambda bi, ci: (bi, jnp.maximum(ci - 1, 0), 0, 0))

    y = pl.pallas_call(
        _conv_kernel,
        grid=(B, rchunks),
        in_specs=[
            pl.BlockSpec((1, 9, IC, OC), lambda bi, ci: (bi, 0, 0, 0)),
            phase_spec, phase_spec, phase_spec, phase_spec,
            halo_spec, halo_spec,
            pl.BlockSpec((1, 1, OC), lambda bi, ci: (bi, 0, 0)),
        ],
        out_specs=pl.BlockSpec((1, R, OW, OC), lambda bi, ci: (bi, ci, 0, 0)),
        out_shape=jax.ShapeDtypeStruct((B, OH, OW, OC), jnp.float32),
        compiler_params=pltpu.CompilerParams(
            dimension_semantics=("parallel", "arbitrary")),
    )(w_comb, p00, p01, p10, p11, p10, p11, b_comb)

    y = jnp.transpose(y, (0, 3, 1, 2))
    return y, loss.reshape(())


# speedup vs baseline: 22.6126x; 1.0627x over previous
import jax
import jax.numpy as jnp
from jax.experimental import pallas as pl
from jax.experimental.pallas import tpu as pltpu

_E = 8
_TOPK = 2


def _pool_kernel(x_ref, out_ref):
    ci = pl.program_id(1)
    w = x_ref.shape[3]
    scale = 1.0 / (w * w)
    s = (jnp.sum(x_ref[0], axis=(1, 2)) * scale).reshape(-1, 1)

    @pl.when(ci == 0)
    def _():
        out_ref[0] = s

    @pl.when(ci > 0)
    def _():
        out_ref[0] = out_ref[0] + s


def _gating_kernel(gx_ref, wg_ref, wf_ref, bias_ref,
                   wcomb_ref, bcomb_ref, loss_ref):
    gx = gx_ref[...]
    wg = wg_ref[...]
    logits = jnp.dot(gx, wg, preferred_element_type=jnp.float32)
    z = logits - jnp.max(logits, axis=1, keepdims=True)
    ez = jnp.exp(z)
    p = ez / jnp.sum(ez, axis=1, keepdims=True)

    iota = jax.lax.broadcasted_iota(jnp.int32, p.shape, 1)
    m1 = jnp.max(p, axis=1, keepdims=True)
    e1 = jnp.min(jnp.where(p == m1, iota, _E), axis=1, keepdims=True)
    p2 = jnp.where(iota == e1, -jnp.inf, p)
    m2 = jnp.max(p2, axis=1, keepdims=True)
    e2 = jnp.min(jnp.where(p2 == m2, iota, _E), axis=1, keepdims=True)
    denom = m1 + m2 + 1e-6
    gates = (jnp.where(iota == e1, m1, 0.0)
             + jnp.where(iota == e2, m2, 0.0)) / denom

    def _cv_sq(v):
        mean = jnp.sum(v, keepdims=True) / _E
        var = jnp.sum((v - mean) ** 2, keepdims=True) / (_E - 1)
        return var / (mean * mean + 1e-10)

    importance = jnp.sum(gates, axis=0, keepdims=True)
    load = jnp.sum((gates > 0).astype(jnp.float32), axis=0, keepdims=True)
    loss_ref[...] = (_cv_sq(importance) + _cv_sq(load)) * 0.01

    wcomb_ref[...] = jnp.dot(gates, wf_ref[...],
                             preferred_element_type=jnp.float32)
    bcomb_ref[...] = jnp.dot(gates, bias_ref[...],
                             preferred_element_type=jnp.float32)


def _shift_col(p):
    return jnp.concatenate(
        [jnp.zeros((p.shape[0], 1, p.shape[2]), p.dtype), p[:, :-1, :]],
        axis=1)


def _shift_row(p, prev_block):
    ci = pl.program_id(1)
    prev_row = prev_block[-1:, :, :]
    prev_row = jnp.where(ci == 0, jnp.zeros_like(prev_row), prev_row)
    return jnp.concatenate([prev_row, p[:-1, :, :]], axis=0)


def _conv_kernel(w_ref, p00_ref, p01_ref, p10_ref, p11_ref,
                 p10h_ref, p11h_ref, b_ref, out_ref):
    R, OW, OC = p00_ref.shape[1], p00_ref.shape[2], out_ref.shape[1]
    p00 = p00_ref[0]
    p01 = p01_ref[0]
    p10 = p10_ref[0]
    p11 = p11_ref[0]
    p10m = _shift_row(p10, p10h_ref[0])
    p11m = _shift_row(p11, p11h_ref[0])
    taps = (
        (_shift_col(p11m), 0), (p10m, 1), (p11m, 2),
        (_shift_col(p01), 3), (p00, 4), (p01, 5),
        (_shift_col(p11), 6), (p10, 7), (p11, 8),
    )
    acc = None
    for patch, t in taps:
        patch = patch.reshape(R * OW, patch.shape[2])
        d = jnp.dot(patch, w_ref[0, t], preferred_element_type=jnp.float32)
        acc = d if acc is None else acc + d
    res = (acc + b_ref[0]).reshape(R, OW, OC)
    out_ref[0] = jnp.transpose(res, (2, 0, 1))


def kernel(x, train, w_gate, w_noise, W, b):
    del train, w_noise
    B, IC, H, Wd = x.shape
    E, OC = W.shape[0], W.shape[1]
    OH, OW = H // 2, Wd // 2

    hchunks = 4
    gate_x = pl.pallas_call(
        _pool_kernel,
        grid=(B, hchunks),
        in_specs=[pl.BlockSpec((1, IC, H // hchunks, Wd),
                               lambda bi, ci: (bi, 0, ci, 0))],
        out_specs=pl.BlockSpec((1, IC, 1), lambda bi, ci: (bi, 0, 0)),
        out_shape=jax.ShapeDtypeStruct((B, IC, 1), jnp.float32),
        compiler_params=pltpu.CompilerParams(
            dimension_semantics=("arbitrary", "arbitrary")),
    )(x)
    gate_x = gate_x.reshape(B, IC)

    W_flat = jnp.transpose(W, (0, 3, 4, 2, 1)).reshape(E, 9 * IC * OC)
    w_comb, b_comb, loss = pl.pallas_call(
        _gating_kernel,
        out_shape=(
            jax.ShapeDtypeStruct((B, 9 * IC * OC), jnp.float32),
            jax.ShapeDtypeStruct((B, OC), jnp.float32),
            jax.ShapeDtypeStruct((1, 1), jnp.float32),
        ),
    )(gate_x, w_gate, W_flat, b)

    w_comb = w_comb.astype(jnp.bfloat16).reshape(B, 9, IC, OC)
    b_comb = b_comb.reshape(B, 1, OC)

    xr = (x.astype(jnp.bfloat16)
          .reshape(B, IC, OH, 2, OW, 2).transpose(0, 3, 5, 2, 4, 1))
    p00 = xr[:, 0, 0]
    p01 = xr[:, 0, 1]
    p10 = xr[:, 1, 0]
    p11 = xr[:, 1, 1]

    rchunks = 7
    R = OH // rchunks
    phase_spec = pl.BlockSpec((1, R, OW, IC), lambda bi, ci: (bi, ci, 0, 0))
    halo_spec = pl.BlockSpec(
        (1, R, OW, IC),
        lambda bi, ci: (bi, jnp.maximum(ci - 1, 0), 0, 0))

    y = pl.pallas_call(
        _conv_kernel,
        grid=(B, rchunks),
        in_specs=[
            pl.BlockSpec((1, 9, IC, OC), lambda bi, ci: (bi, 0, 0, 0)),
            phase_spec, phase_spec, phase_spec, phase_spec,
            halo_spec, halo_spec,
            pl.BlockSpec((1, 1, OC), lambda bi, ci: (bi, 0, 0)),
        ],
        out_specs=pl.BlockSpec((1, OC, R, OW), lambda bi, ci: (bi, 0, ci, 0)),
        out_shape=jax.ShapeDtypeStruct((B, OC, OH, OW), jnp.float32),
        compiler_params=pltpu.CompilerParams(
            dimension_semantics=("parallel", "arbitrary")),
    )(w_comb, p00, p01, p10, p11, p10, p11, b_comb)

    return y, loss.reshape(())


# speedup vs baseline: 25.6041x; 1.1323x over previous
import jax
import jax.numpy as jnp
from jax.experimental import pallas as pl
from jax.experimental.pallas import tpu as pltpu

_E = 8
_TOPK = 2


def _pool_kernel(x_ref, out_ref):
    ci = pl.program_id(1)
    w = x_ref.shape[3]
    scale = 1.0 / (w * w)
    s = (jnp.sum(x_ref[0], axis=(1, 2)) * scale).reshape(-1, 1)

    @pl.when(ci == 0)
    def _():
        out_ref[0] = s

    @pl.when(ci > 0)
    def _():
        out_ref[0] = out_ref[0] + s


def _gating_kernel(gx_ref, wg_ref, wf_ref, bias_ref,
                   wcomb_ref, bcomb_ref, loss_ref):
    gx = gx_ref[...]
    wg = wg_ref[...]
    logits = jnp.dot(gx, wg, preferred_element_type=jnp.float32)
    z = logits - jnp.max(logits, axis=1, keepdims=True)
    ez = jnp.exp(z)
    p = ez / jnp.sum(ez, axis=1, keepdims=True)

    iota = jax.lax.broadcasted_iota(jnp.int32, p.shape, 1)
    m1 = jnp.max(p, axis=1, keepdims=True)
    e1 = jnp.min(jnp.where(p == m1, iota, _E), axis=1, keepdims=True)
    p2 = jnp.where(iota == e1, -jnp.inf, p)
    m2 = jnp.max(p2, axis=1, keepdims=True)
    e2 = jnp.min(jnp.where(p2 == m2, iota, _E), axis=1, keepdims=True)
    denom = m1 + m2 + 1e-6
    gates = (jnp.where(iota == e1, m1, 0.0)
             + jnp.where(iota == e2, m2, 0.0)) / denom

    def _cv_sq(v):
        mean = jnp.sum(v, keepdims=True) / _E
        var = jnp.sum((v - mean) ** 2, keepdims=True) / (_E - 1)
        return var / (mean * mean + 1e-10)

    importance = jnp.sum(gates, axis=0, keepdims=True)
    load = jnp.sum((gates > 0).astype(jnp.float32), axis=0, keepdims=True)
    loss_ref[...] = (_cv_sq(importance) + _cv_sq(load)) * 0.01

    wcomb_ref[...] = jnp.dot(gates, wf_ref[...],
                             preferred_element_type=jnp.float32)
    bcomb_ref[...] = jnp.dot(gates, bias_ref[...],
                             preferred_element_type=jnp.float32)


def _shift_col(p):
    return jnp.concatenate(
        [jnp.zeros((p.shape[0], 1, p.shape[2]), p.dtype), p[:, :-1, :]],
        axis=1)


def _shift_row(p, prev_block):
    ci = pl.program_id(1)
    prev_row = prev_block[-1:, :, :]
    prev_row = jnp.where(ci == 0, jnp.zeros_like(prev_row), prev_row)
    return jnp.concatenate([prev_row, p[:-1, :, :]], axis=0)


def _conv_kernel(w_ref, p00_ref, p01_ref, p10_ref, p11_ref,
                 p10h_ref, p11h_ref, b_ref, out_ref):
    R, OW, OC = p00_ref.shape[3], p00_ref.shape[4], out_ref.shape[1]
    p00 = p00_ref[0, 0, 0]
    p01 = p01_ref[0, 0, 0]
    p10 = p10_ref[0, 0, 0]
    p11 = p11_ref[0, 0, 0]
    p10m = _shift_row(p10, p10h_ref[0, 0, 0])
    p11m = _shift_row(p11, p11h_ref[0, 0, 0])
    taps = (
        (_shift_col(p11m), 0), (p10m, 1), (p11m, 2),
        (_shift_col(p01), 3), (p00, 4), (p01, 5),
        (_shift_col(p11), 6), (p10, 7), (p11, 8),
    )
    acc = None
    for patch, t in taps:
        patch = patch.reshape(R * OW, patch.shape[2])
        d = jnp.dot(patch, w_ref[0, t], preferred_element_type=jnp.float32)
        acc = d if acc is None else acc + d
    res = (acc + b_ref[0]).reshape(R, OW, OC)
    out_ref[0] = jnp.transpose(res, (2, 0, 1))


def kernel(x, train, w_gate, w_noise, W, b):
    del train, w_noise
    B, IC, H, Wd = x.shape
    E, OC = W.shape[0], W.shape[1]
    OH, OW = H // 2, Wd // 2

    hchunks = 4
    gate_x = pl.pallas_call(
        _pool_kernel,
        grid=(B, hchunks),
        in_specs=[pl.BlockSpec((1, IC, H // hchunks, Wd),
                               lambda bi, ci: (bi, 0, ci, 0))],
        out_specs=pl.BlockSpec((1, IC, 1), lambda bi, ci: (bi, 0, 0)),
        out_shape=jax.ShapeDtypeStruct((B, IC, 1), jnp.float32),
        compiler_params=pltpu.CompilerParams(
            dimension_semantics=("arbitrary", "arbitrary")),
    )(x)
    gate_x = gate_x.reshape(B, IC)

    W_flat = jnp.transpose(W, (0, 3, 4, 2, 1)).reshape(E, 9 * IC * OC)
    w_comb, b_comb, loss = pl.pallas_call(
        _gating_kernel,
        out_shape=(
            jax.ShapeDtypeStruct((B, 9 * IC * OC), jnp.float32),
            jax.ShapeDtypeStruct((B, OC), jnp.float32),
            jax.ShapeDtypeStruct((1, 1), jnp.float32),
        ),
    )(gate_x, w_gate, W_flat, b)

    w_comb = w_comb.astype(jnp.bfloat16).reshape(B, 9, IC, OC)
    b_comb = b_comb.reshape(B, 1, OC)

    xr = (x.astype(jnp.bfloat16)
          .reshape(B, IC, OH, 2, OW, 2).transpose(0, 3, 5, 2, 4, 1))

    rchunks = 7
    R = OH // rchunks
    blk = (1, 1, 1, R, OW, IC)

    def _phase(ry, rx):
        return pl.BlockSpec(blk, lambda bi, ci: (bi, ry, rx, ci, 0, 0))

    def _halo(ry, rx):
        return pl.BlockSpec(
            blk, lambda bi, ci: (bi, ry, rx, jnp.maximum(ci - 1, 0), 0, 0))

    y = pl.pallas_call(
        _conv_kernel,
        grid=(B, rchunks),
        in_specs=[
            pl.BlockSpec((1, 9, IC, OC), lambda bi, ci: (bi, 0, 0, 0)),
            _phase(0, 0), _phase(0, 1), _phase(1, 0), _phase(1, 1),
            _halo(1, 0), _halo(1, 1),
            pl.BlockSpec((1, 1, OC), lambda bi, ci: (bi, 0, 0)),
        ],
        out_specs=pl.BlockSpec((1, OC, R, OW), lambda bi, ci: (bi, 0, ci, 0)),
        out_shape=jax.ShapeDtypeStruct((B, OC, OH, OW), jnp.float32),
        compiler_params=pltpu.CompilerParams(
            dimension_semantics=("parallel", "arbitrary")),
    )(w_comb, xr, xr, xr, xr, xr, xr, b_comb)

    return y, loss.reshape(())


# speedup vs baseline: 28.7753x; 1.1239x over previous
import jax
import jax.numpy as jnp
from jax.experimental import pallas as pl
from jax.experimental.pallas import tpu as pltpu

_E = 8
_TOPK = 2


def _pool_kernel(x_ref, out_ref, xr_ref):
    ci = pl.program_id(1)
    ic, rows, w = x_ref.shape[1], x_ref.shape[2], x_ref.shape[3]
    scale = 1.0 / (w * w)
    xv = x_ref[0]
    s = (jnp.sum(xv, axis=(1, 2)) * scale).reshape(-1, 1)

    @pl.when(ci == 0)
    def _():
        out_ref[0] = s

    @pl.when(ci > 0)
    def _():
        out_ref[0] = out_ref[0] + s

    xr_ref[0] = xv.astype(jnp.bfloat16)


def _gating_kernel(gx_ref, wg_ref, wf_ref, bias_ref,
                   wcomb_ref, bcomb_ref, loss_ref):
    gx = gx_ref[...]
    wg = wg_ref[...]
    logits = jnp.dot(gx, wg, preferred_element_type=jnp.float32)
    z = logits - jnp.max(logits, axis=1, keepdims=True)
    ez = jnp.exp(z)
    p = ez / jnp.sum(ez, axis=1, keepdims=True)

    iota = jax.lax.broadcasted_iota(jnp.int32, p.shape, 1)
    m1 = jnp.max(p, axis=1, keepdims=True)
    e1 = jnp.min(jnp.where(p == m1, iota, _E), axis=1, keepdims=True)
    p2 = jnp.where(iota == e1, -jnp.inf, p)
    m2 = jnp.max(p2, axis=1, keepdims=True)
    e2 = jnp.min(jnp.where(p2 == m2, iota, _E), axis=1, keepdims=True)
    denom = m1 + m2 + 1e-6
    gates = (jnp.where(iota == e1, m1, 0.0)
             + jnp.where(iota == e2, m2, 0.0)) / denom

    def _cv_sq(v):
        mean = jnp.sum(v, keepdims=True) / _E
        var = jnp.sum((v - mean) ** 2, keepdims=True) / (_E - 1)
        return var / (mean * mean + 1e-10)

    importance = jnp.sum(gates, axis=0, keepdims=True)
    load = jnp.sum((gates > 0).astype(jnp.float32), axis=0, keepdims=True)
    loss_ref[...] = (_cv_sq(importance) + _cv_sq(load)) * 0.01

    wcomb_ref[...] = jnp.dot(gates, wf_ref[...],
                             preferred_element_type=jnp.float32)
    bcomb_ref[...] = jnp.dot(gates, bias_ref[...],
                             preferred_element_type=jnp.float32)


def _shift_col(p):
    return jnp.concatenate(
        [jnp.zeros((p.shape[0], 1, p.shape[2]), p.dtype), p[:, :-1, :]],
        axis=1)


def _shift_row(p, prev_block):
    ci = pl.program_id(1)
    prev_row = prev_block[-1:, :, :]
    prev_row = jnp.where(ci == 0, jnp.zeros_like(prev_row), prev_row)
    return jnp.concatenate([prev_row, p[:-1, :, :]], axis=0)


def _conv_kernel(w_ref, p00_ref, p01_ref, p10_ref, p11_ref,
                 p10h_ref, p11h_ref, b_ref, out_ref):
    R, OW, OC = p00_ref.shape[3], p00_ref.shape[4], out_ref.shape[1]
    p00 = p00_ref[0, 0, 0]
    p01 = p01_ref[0, 0, 0]
    p10 = p10_ref[0, 0, 0]
    p11 = p11_ref[0, 0, 0]
    p10m = _shift_row(p10, p10h_ref[0, 0, 0])
    p11m = _shift_row(p11, p11h_ref[0, 0, 0])
    taps = (
        (_shift_col(p11m), 0), (p10m, 1), (p11m, 2),
        (_shift_col(p01), 3), (p00, 4), (p01, 5),
        (_shift_col(p11), 6), (p10, 7), (p11, 8),
    )
    acc = None
    for patch, t in taps:
        patch = patch.reshape(R * OW, patch.shape[2])
        d = jnp.dot(patch, w_ref[0, t], preferred_element_type=jnp.float32)
        acc = d if acc is None else acc + d
    res = (acc + b_ref[0]).reshape(R, OW, OC)
    out_ref[0] = jnp.transpose(res, (2, 0, 1))


def kernel(x, train, w_gate, w_noise, W, b):
    del train, w_noise
    B, IC, H, Wd = x.shape
    E, OC = W.shape[0], W.shape[1]
    OH, OW = H // 2, Wd // 2

    hchunks = 4
    HB = H // hchunks
    gate_x, xb = pl.pallas_call(
        _pool_kernel,
        grid=(B, hchunks),
        in_specs=[pl.BlockSpec((1, IC, HB, Wd),
                               lambda bi, ci: (bi, 0, ci, 0))],
        out_specs=(
            pl.BlockSpec((1, IC, 1), lambda bi, ci: (bi, 0, 0)),
            pl.BlockSpec((1, IC, HB, Wd), lambda bi, ci: (bi, 0, ci, 0)),
        ),
        out_shape=(
            jax.ShapeDtypeStruct((B, IC, 1), jnp.float32),
            jax.ShapeDtypeStruct((B, IC, H, Wd), jnp.bfloat16),
        ),
        compiler_params=pltpu.CompilerParams(
            dimension_semantics=("arbitrary", "arbitrary")),
    )(x)
    gate_x = gate_x.reshape(B, IC)

    xr = xb.reshape(B, IC, OH, 2, OW, 2).transpose(0, 3, 5, 2, 4, 1)

    W_flat = jnp.transpose(W, (0, 3, 4, 2, 1)).reshape(E, 9 * IC * OC)
    w_comb, b_comb, loss = pl.pallas_call(
        _gating_kernel,
        out_shape=(
            jax.ShapeDtypeStruct((B, 9 * IC * OC), jnp.float32),
            jax.ShapeDtypeStruct((B, OC), jnp.float32),
            jax.ShapeDtypeStruct((1, 1), jnp.float32),
        ),
    )(gate_x, w_gate, W_flat, b)

    w_comb = w_comb.astype(jnp.bfloat16).reshape(B, 9, IC, OC)
    b_comb = b_comb.reshape(B, 1, OC)

    rchunks = 7
    R = OH // rchunks
    blk = (1, 1, 1, R, OW, IC)

    def _phase(ry, rx):
        return pl.BlockSpec(blk, lambda bi, ci: (bi, ry, rx, ci, 0, 0))

    def _halo(ry, rx):
        return pl.BlockSpec(
            blk, lambda bi, ci: (bi, ry, rx, jnp.maximum(ci - 1, 0), 0, 0))

    y = pl.pallas_call(
        _conv_kernel,
        grid=(B, rchunks),
        in_specs=[
            pl.BlockSpec((1, 9, IC, OC), lambda bi, ci: (bi, 0, 0, 0)),
            _phase(0, 0), _phase(0, 1), _phase(1, 0), _phase(1, 1),
            _halo(1, 0), _halo(1, 1),
            pl.BlockSpec((1, 1, OC), lambda bi, ci: (bi, 0, 0)),
        ],
        out_specs=pl.BlockSpec((1, OC, R, OW), lambda bi, ci: (bi, 0, ci, 0)),
        out_shape=jax.ShapeDtypeStruct((B, OC, OH, OW), jnp.float32),
        compiler_params=pltpu.CompilerParams(
            dimension_semantics=("parallel", "arbitrary")),
    )(w_comb, xr, xr, xr, xr, xr, xr, b_comb)

    return y, loss.reshape(())
